# baseline (speedup 1.0000x reference)
import functools

import jax
import jax.numpy as jnp
from jax import lax
from jax.experimental import pallas as pl
from jax.experimental.pallas import tpu as pltpu
from jax.experimental.pallas import tpu_sc as plsc

_DIM = 1024
_ROWS = 8192
_NC, _NS = 2, 16
_NW = _NC * _NS
_SC_ROWS = 3072
_ROWS_PER_W = _SC_ROWS // _NW
_CHUNK = 32
_NCHUNK = _ROWS_PER_W // _CHUNK
_NBUF = 3

_TC_ROWS = _ROWS - _SC_ROWS
_BLK = 512


@functools.partial(
    pl.kernel,
    mesh=plsc.VectorSubcoreMesh(core_axis_name="c", subcore_axis_name="s"),
    out_type=jax.ShapeDtypeStruct((_SC_ROWS, _DIM), jnp.float32),
    scratch_types=(
        [pltpu.VMEM((_CHUNK, _DIM), jnp.float32) for _ in range(_NBUF)]
        + [pltpu.SemaphoreType.DMA for _ in range(2 * _NBUF)]
    ),
)
def _sc_copy(emb_hbm, out_hbm, *scratch):
    bufs = scratch[:_NBUF]
    gsems = scratch[_NBUF:2 * _NBUF]
    ssems = scratch[2 * _NBUF:]
    wid = lax.axis_index("s") * _NC + lax.axis_index("c")
    base = wid * _ROWS_PER_W

    def gather(i):
        b = i % _NBUF
        return pltpu.make_async_copy(
            emb_hbm.at[pl.ds(base + i * _CHUNK, _CHUNK)], bufs[b], gsems[b])

    def scatter(i):
        b = i % _NBUF
        return pltpu.make_async_copy(
            bufs[b], out_hbm.at[pl.ds(base + i * _CHUNK, _CHUNK)], ssems[b])

    for i in range(_NCHUNK):
        gather(i).start()
    for i in range(_NCHUNK):
        gather(i).wait()
        scatter(i).start()
    for i in range(_NCHUNK):
        scatter(i).wait()


def _tc_body(i_ref, o_ref):
    o_ref[...] = i_ref[...]


def _tc_copy(emb_weight):
    return pl.pallas_call(
        _tc_body,
        grid=(_TC_ROWS // _BLK,),
        in_specs=[pl.BlockSpec((_BLK, _DIM), lambda i: (i + _SC_ROWS // _BLK, 0))],
        out_specs=pl.BlockSpec((_BLK, _DIM), lambda i: (i, 0)),
        out_shape=jax.ShapeDtypeStruct((_TC_ROWS, _DIM), jnp.float32),
    )(emb_weight)


def kernel(x, emb_weight):
    del x
    sc_part = _sc_copy(emb_weight)
    tc_part = _tc_copy(emb_weight)
    return jnp.concatenate([sc_part, tc_part], axis=0)[None, :, :]


# speedup vs baseline: 1.3496x; 1.3496x over previous
import functools

import jax
import jax.numpy as jnp
from jax import lax
from jax.experimental import pallas as pl
from jax.experimental.pallas import tpu as pltpu
from jax.experimental.pallas import tpu_sc as plsc

_DIM = 1024
_ROWS = 8192
_NC, _NS = 2, 16
_ROWS_PER_SC = _ROWS // _NC

_SP_ROWS = 2048
_SP_CHUNK = 256
_SP_NCHUNK = _SP_ROWS // _SP_CHUNK
_SP_NBUF = 2

_TS_ROWS = _ROWS_PER_SC - _SP_ROWS
_TS_PER_TILE = _TS_ROWS // _NS
_TS_CHUNK = 32
_TS_NCHUNK = _TS_PER_TILE // _TS_CHUNK
_TS_NBUF = 3


@functools.partial(
    pl.kernel,
    mesh=plsc.VectorSubcoreMesh(core_axis_name="c", subcore_axis_name="s"),
    out_type=jax.ShapeDtypeStruct((_ROWS, _DIM), jnp.float32),
    scratch_types=(
        [pltpu.VMEM((_TS_CHUNK, _DIM), jnp.float32) for _ in range(_TS_NBUF)]
        + [pltpu.VMEM_SHARED((_SP_CHUNK, _DIM), jnp.float32) for _ in range(_SP_NBUF)]
        + [pltpu.SemaphoreType.DMA for _ in range(2 * _TS_NBUF + 2 * _SP_NBUF)]
    ),
)
def _sc_copy(emb_hbm, out_hbm, *scratch):
    ts_bufs = scratch[:_TS_NBUF]
    sp_bufs = scratch[_TS_NBUF:_TS_NBUF + _SP_NBUF]
    sems = scratch[_TS_NBUF + _SP_NBUF:]
    ts_gsems = sems[:_TS_NBUF]
    ts_ssems = sems[_TS_NBUF:2 * _TS_NBUF]
    sp_gsems = sems[2 * _TS_NBUF:2 * _TS_NBUF + _SP_NBUF]
    sp_ssems = sems[2 * _TS_NBUF + _SP_NBUF:]

    cid = lax.axis_index("c")
    sid = lax.axis_index("s")
    sc_base = cid * _ROWS_PER_SC

    def sp_gather(i):
        b = i % _SP_NBUF
        return pltpu.make_async_copy(
            emb_hbm.at[pl.ds(sc_base + i * _SP_CHUNK, _SP_CHUNK)],
            sp_bufs[b], sp_gsems[b])

    def sp_scatter(i):
        b = i % _SP_NBUF
        return pltpu.make_async_copy(
            sp_bufs[b],
            out_hbm.at[pl.ds(sc_base + i * _SP_CHUNK, _SP_CHUNK)],
            sp_ssems[b])

    ts_base = sc_base + _SP_ROWS + sid * _TS_PER_TILE

    def ts_gather(i):
        b = i % _TS_NBUF
        return pltpu.make_async_copy(
            emb_hbm.at[pl.ds(ts_base + i * _TS_CHUNK, _TS_CHUNK)],
            ts_bufs[b], ts_gsems[b])

    def ts_scatter(i):
        b = i % _TS_NBUF
        return pltpu.make_async_copy(
            ts_bufs[b],
            out_hbm.at[pl.ds(ts_base + i * _TS_CHUNK, _TS_CHUNK)],
            ts_ssems[b])

    @pl.when(sid == 0)
    def _():
        sp_gather(0).start()

    ts_gather(0).start()
    ts_gather(1).start()

    for k in range(_SP_NCHUNK):
        if k < _TS_NCHUNK:
            i = k
            ts_gather(i).wait()
            ts_scatter(i).start()
            if i + 2 < _TS_NCHUNK:
                if i >= 1:
                    ts_scatter(i - 1).wait()
                ts_gather(i + 2).start()

        @pl.when(sid == 0)
        def _(k=k):
            sp_gather(k).wait()
            sp_scatter(k).start()
            if k + 1 < _SP_NCHUNK:
                if k >= 1:
                    sp_scatter(k - 1).wait()
                sp_gather(k + 1).start()

    @pl.when(sid == 0)
    def _():
        for i in range(_SP_NCHUNK - _SP_NBUF, _SP_NCHUNK):
            sp_scatter(i).wait()

    for i in range(max(0, _TS_NCHUNK - _TS_NBUF), _TS_NCHUNK):
        ts_scatter(i).wait()


def kernel(x, emb_weight):
    del x
    return _sc_copy(emb_weight)[None, :, :]


# speedup vs baseline: 1.4408x; 1.0676x over previous
import functools

import jax
import jax.numpy as jnp
from jax import lax
from jax.experimental import pallas as pl
from jax.experimental.pallas import tpu as pltpu
from jax.experimental.pallas import tpu_sc as plsc

_DIM = 1024
_ROWS = 8192
_NC, _NS = 2, 16
_NW = _NC * _NS
_ROWS_PER_W = _ROWS // _NW
_CHUNK = 16
_NCHUNK = _ROWS_PER_W // _CHUNK
_NBUF = 7
_G = 4


@functools.partial(
    pl.kernel,
    mesh=plsc.VectorSubcoreMesh(core_axis_name="c", subcore_axis_name="s"),
    out_type=jax.ShapeDtypeStruct((_ROWS, _DIM), jnp.float32),
    scratch_types=(
        [pltpu.VMEM((_CHUNK, _DIM), jnp.float32) for _ in range(_NBUF)]
        + [pltpu.SemaphoreType.DMA for _ in range(2 * _NBUF)]
    ),
)
def _sc_copy(emb_hbm, out_hbm, *scratch):
    bufs = scratch[:_NBUF]
    gsems = scratch[_NBUF:2 * _NBUF]
    ssems = scratch[2 * _NBUF:]
    wid = lax.axis_index("s") * _NC + lax.axis_index("c")
    base = wid * _ROWS_PER_W

    def gather(i):
        b = i % _NBUF
        return pltpu.make_async_copy(
            emb_hbm.at[pl.ds(base + i * _CHUNK, _CHUNK)], bufs[b], gsems[b])

    def scatter(i):
        b = i % _NBUF
        return pltpu.make_async_copy(
            bufs[b], out_hbm.at[pl.ds(base + i * _CHUNK, _CHUNK)], ssems[b])

    for i in range(_G):
        gather(i).start()
    for i in range(_NCHUNK):
        gather(i).wait()
        scatter(i).start()
        j = i + _G
        if j < _NCHUNK:
            if j - _NBUF >= 0:
                scatter(j - _NBUF).wait()
            gather(j).start()
    for i in range(_NCHUNK - _NBUF, _NCHUNK):
        scatter(i).wait()


def kernel(x, emb_weight):
    del x
    return _sc_copy(emb_weight)[None, :, :]
